# baseline (speedup 1.0000x reference)
import functools

import jax
import jax.numpy as jnp
from jax import lax
from jax.experimental import pallas as pl
from jax.experimental.pallas import tpu as pltpu
from jax.experimental.pallas import tpu_sc as plsc

N_LNC, N_MI, N_M = 10000, 5000, 20000
N_TOT = N_LNC + N_MI + N_M
OFF = {"lnc": 0, "mi": N_LNC, "m": N_LNC + N_MI}
FEAT = 256
OUT = 128
E = 50000
NB = 4
NR = 6

NCORES = 2
NS = 16
W = 128
NPCOL = FEAT // W
CHUNK = 128
EPT = 3200
E_PAD = NS * EPT
NCHUNK = EPT // CHUNK
NRACC = 10112
RPT = NRACC // NS
NSEG = 4
SEGMENTS = (
    dict(rels=(1, 5), srct=("mi", "m"), dst_lo=0, n=N_LNC),
    dict(rels=(0, 3), srct=("lnc", "m"), dst_lo=0, n=N_MI),
    dict(rels=(2, 4), srct=("mi", "lnc"), dst_lo=0, n=10000),
    dict(rels=(2, 4), srct=("mi", "lnc"), dst_lo=10000, n=10000),
)
R_BLK = 1000
SEG_STARTS = (10, 15, 25)


def _seg_body(with_deg, *refs):
    if with_deg:
        (table, ones_hbm, srcidx, dstidx, zeros_hbm,
         s_out, deg_out, acc, srci_all, dsti_all, rows0, rows1,
         gsem0, gsem1, zsem) = refs
    else:
        (table, srcidx, dstidx, zeros_hbm,
         s_out, acc, srci_all, dsti_all, rows0, rows1, gsem0, gsem1,
         zsem) = refs
    c = lax.axis_index("c")
    s = lax.axis_index("s")
    rbase = s * RPT

    def zero_acc():
        pltpu.async_copy(zeros_hbm, acc.at[pl.ds(rbase, RPT)], zsem)

    def zero_wait():
        pltpu.make_async_copy(zeros_hbm, acc.at[pl.ds(rbase, RPT)],
                              zsem).wait()

    def run_pass(src_slab, dst_slab, out_at):
        zero_acc()
        pltpu.sync_copy(src_slab, srci_all)
        pltpu.sync_copy(dst_slab, dsti_all)

        def issue(j, rows, sem):
            pltpu.async_copy(table.at[srci_all.at[j]], rows, sem)

        def wait_scat(j, rows, sem):
            pltpu.make_async_copy(table.at[srci_all.at[j]], rows, sem).wait()
            pltpu.sync_copy(rows, acc.at[dsti_all.at[j]], add=True)

        issue(0, rows0, gsem0)
        issue(1, rows1, gsem1)
        zero_wait()
        plsc.subcore_barrier()
        assert NCHUNK % 2 == 1 and NCHUNK >= 3

        def pbody(t, carry):
            j0 = 2 * t
            wait_scat(j0, rows0, gsem0)
            issue(j0 + 2, rows0, gsem0)
            wait_scat(j0 + 1, rows1, gsem1)
            issue(j0 + 3, rows1, gsem1)
            return carry

        lax.fori_loop(0, (NCHUNK - 3) // 2, pbody, 0)
        wait_scat(NCHUNK - 3, rows0, gsem0)
        issue(NCHUNK - 1, rows0, gsem0)
        wait_scat(NCHUNK - 2, rows1, gsem1)
        wait_scat(NCHUNK - 1, rows0, gsem0)
        plsc.subcore_barrier()
        pltpu.sync_copy(acc.at[pl.ds(rbase, RPT)], out_at)

    def run_deg_pass(dst_slab, out_at):
        zero_acc()
        pltpu.sync_copy(dst_slab, dsti_all)
        pltpu.sync_copy(ones_hbm, rows0)
        zero_wait()
        plsc.subcore_barrier()

        def dbody(j, carry):
            pltpu.async_copy(rows0, acc.at[dsti_all.at[j]], gsem0, add=True)
            return carry

        lax.fori_loop(0, NCHUNK, dbody, 0)

        def dwait(j, carry):
            pltpu.make_async_copy(rows0, acc.at[dsti_all.at[j]],
                                  gsem0).wait()
            return carry

        lax.fori_loop(0, NCHUNK, dwait, 0)
        plsc.subcore_barrier()
        pltpu.sync_copy(acc.at[pl.ds(rbase, RPT)], out_at)

    for g in range(NSEG):
        for p in range(NPCOL):
            run_pass(
                srcidx.at[c, g, p, s],
                dstidx.at[c, g, s],
                s_out.at[c, g, p, pl.ds(rbase, RPT)],
            )
        if with_deg:
            run_deg_pass(
                dstidx.at[c, g, s],
                deg_out.at[c, g, pl.ds(rbase, RPT)],
            )


@functools.cache
def _make_seg_kernel(with_deg):
    out_type = [jax.ShapeDtypeStruct((NCORES, NSEG, NPCOL, NRACC, W),
                                     jnp.float32)]
    if with_deg:
        out_type.append(
            jax.ShapeDtypeStruct((NCORES, NSEG, NRACC, W), jnp.float32))
    scratch = (
        pltpu.VMEM_SHARED((NRACC, W), jnp.float32),
        pltpu.VMEM((NCHUNK, CHUNK), jnp.int32),
        pltpu.VMEM((NCHUNK, CHUNK), jnp.int32),
        pltpu.VMEM((CHUNK, W), jnp.float32),
        pltpu.VMEM((CHUNK, W), jnp.float32),
        pltpu.SemaphoreType.DMA,
        pltpu.SemaphoreType.DMA,
        pltpu.SemaphoreType.DMA,
    )
    mesh = plsc.VectorSubcoreMesh(core_axis_name="c", subcore_axis_name="s")
    body = functools.partial(_seg_body, with_deg)
    return pl.kernel(body, out_type=tuple(out_type), mesh=mesh,
                     scratch_types=scratch,
                     name="segsum_deg" if with_deg else "segsum")


def _combine_w(coeff1, basis1, coeff2, basis2):
    def body(c1_ref, b1_ref, c2_ref, b2_ref, w1_ref, w2_ref):
        for c_ref, b_ref, w_ref in ((c1_ref, b1_ref, w1_ref),
                                    (c2_ref, b2_ref, w2_ref)):
            for r in range(NR):
                acc = c_ref[r, 0] * b_ref[0]
                for b in range(1, NB):
                    acc = acc + c_ref[r, b] * b_ref[b]
                w_ref[r] = acc

    out = jax.ShapeDtypeStruct((NR, FEAT, FEAT), jnp.float32)
    return pl.pallas_call(
        body,
        in_specs=[pl.BlockSpec(memory_space=pltpu.SMEM),
                  pl.BlockSpec((NB, FEAT, FEAT), lambda: (0, 0, 0)),
                  pl.BlockSpec(memory_space=pltpu.SMEM),
                  pl.BlockSpec((NB, FEAT, FEAT), lambda: (0, 0, 0))],
        out_specs=[pl.BlockSpec((NR, FEAT, FEAT), lambda: (0, 0, 0))] * 2,
        out_shape=[out, out],
    )(coeff1, basis1, coeff2, basis2)


def _seg_of(i):
    s = jnp.int32(0)
    for b in SEG_STARTS:
        s = s + (i >= b).astype(jnp.int32)
    return s


def _blk_of(i):
    off = jnp.int32(0)
    starts = (0,) + SEG_STARTS
    for k in range(1, NSEG):
        off = off + (i >= starts[k]).astype(jnp.int32) * (
            starts[k] - starts[k - 1])
    return i - off


def _tc_layer(h_all, s_all, deg_all, wp_all, loop_w, bias, dec):
    def bdot(x, y):
        return jnp.dot(x.astype(jnp.bfloat16), y.astype(jnp.bfloat16),
                       preferred_element_type=jnp.float32)

    def body(h_ref, s_ref, deg_ref, wp_ref, loop_ref, bias_ref, *rest):
        o_ref = rest[-1]
        acc = bdot(h_ref[...], loop_ref[...])
        for a in range(2):
            d = deg_ref[a, 0][:, 0:1]
            inv = 1.0 / jnp.maximum(d, 1.0)
            sa = jnp.concatenate([s_ref[a, 0, p] for p in range(NPCOL)],
                                 axis=1)
            acc = acc + bdot(sa * inv, wp_ref[0, a])
        x = jnp.maximum(acc + bias_ref[...], 0.0)
        if dec is None:
            o_ref[...] = x
        else:
            dw1_ref, db1_ref, dw2_ref, db2_ref = rest[:-1]
            y = jnp.maximum(bdot(x, dw1_ref[...]) + db1_ref[...], 0.0)
            o_ref[...] = bdot(y, dw2_ref[...]) + db2_ref[...]

    in_specs = [
        pl.BlockSpec((R_BLK, FEAT), lambda i: (i, 0)),
        pl.BlockSpec((NCORES, 1, NPCOL, R_BLK, W),
                     lambda i: (0, _seg_of(i), 0, _blk_of(i), 0)),
        pl.BlockSpec((NCORES, 1, R_BLK, 8),
                     lambda i: (0, _seg_of(i), _blk_of(i), 0)),
        pl.BlockSpec((1, NCORES, FEAT, FEAT),
                     lambda i: (_seg_of(i), 0, 0, 0)),
        pl.BlockSpec((FEAT, FEAT), lambda i: (0, 0)),
        pl.BlockSpec((1, FEAT), lambda i: (0, 0)),
    ]
    args = [h_all, s_all, deg_all, wp_all, loop_w, bias]
    out_w = FEAT
    if dec is not None:
        dw1, db1, dw2, db2 = dec
        in_specs += [
            pl.BlockSpec((FEAT, FEAT), lambda i: (0, 0)),
            pl.BlockSpec((1, FEAT), lambda i: (0, 0)),
            pl.BlockSpec((FEAT, OUT), lambda i: (0, 0)),
            pl.BlockSpec((1, OUT), lambda i: (0, 0)),
        ]
        args += [dw1, db1, dw2, db2]
        out_w = OUT

    return pl.pallas_call(
        body,
        grid=(N_TOT // R_BLK,),
        in_specs=in_specs,
        out_specs=pl.BlockSpec((R_BLK, out_w), lambda i: (i, 0)),
        out_shape=jax.ShapeDtypeStruct((N_TOT, out_w), jnp.float32),
    )(*args)


def kernel(h_lnc, h_mi, h_m, src0, dst0, src1, dst1, src2, dst2, src3, dst3,
           src4, dst4, src5, dst5, basis1, coeff1, loop1, bias1, basis2,
           coeff2, loop2, bias2, dec_W1, dec_b1, dec_W2, dec_b2):
    srcs = [src0, src1, src2, src3, src4, src5]
    dsts = [dst0, dst1, dst2, dst3, dst4, dst5]

    W1s, W2s = _combine_w(coeff1, basis1, coeff2, basis2)

    epad = E_PAD - E
    eidx = jnp.arange(E_PAD, dtype=jnp.int32)
    spread = eidx % 2048
    trash = (NRACC - 16) + (eidx & 15)
    src_segs, dst_segs = [], []
    for seg in SEGMENTS:
        src_pair, dst_pair = [], []
        for r, st in zip(seg["rels"], seg["srct"]):
            sp = jnp.concatenate(
                [srcs[r] + OFF[st], jnp.zeros((epad,), jnp.int32)])
            dp = jnp.concatenate(
                [dsts[r], jnp.full((epad,), jnp.int32(1 << 28))])
            lo = seg["dst_lo"]
            ok = (dp >= lo) & (dp < lo + seg["n"])
            dst_pair.append(jnp.where(ok, dp - lo, trash))
            src_pair.append(jnp.where(ok, sp, spread) * NPCOL)
        src_segs.append(jnp.stack(src_pair))
        dst_segs.append(jnp.stack(dst_pair))
    src_base = jnp.stack(src_segs, axis=1)
    srcidx = (jnp.stack([src_base + p for p in range(NPCOL)], axis=2)
              .reshape(NCORES, NSEG, NPCOL, NS, NCHUNK, CHUNK))
    dstidx = jnp.stack(dst_segs, axis=1).reshape(
        NCORES, NSEG, NS, NCHUNK, CHUNK)

    ones_tab = jnp.ones((CHUNK, W), jnp.float32)
    zeros_w = jnp.zeros((RPT, W), jnp.float32)

    wp_all = {}
    for li, Ws in ((0, W1s), (1, W2s)):
        wp_all[li] = jnp.stack(
            [jnp.stack([Ws[seg["rels"][0]], Ws[seg["rels"][1]]])
             for seg in SEGMENTS])

    h_all = jnp.concatenate([h_lnc, h_mi, h_m], axis=0)
    deg_all = None
    for layer in (0, 1):
        table = h_all.reshape(N_TOT * NPCOL, W)
        if layer == 0:
            s_all, deg_fat = _make_seg_kernel(True)(
                table, ones_tab, srcidx, dstidx, zeros_w)
            deg_all = deg_fat[:, :, :, :8]
        else:
            (s_all,) = _make_seg_kernel(False)(table, srcidx, dstidx, zeros_w)
        loop_w = loop1 if layer == 0 else loop2
        bias = (bias1 if layer == 0 else bias2).reshape(1, FEAT)
        dec = (None if layer == 0 else
               (dec_W1, dec_b1.reshape(1, FEAT), dec_W2,
                dec_b2.reshape(1, OUT)))
        h_all = _tc_layer(h_all, s_all, deg_all, wp_all[layer], loop_w,
                          bias, dec)
    return h_all


# speedup vs baseline: 1.0491x; 1.0491x over previous
import functools

import jax
import jax.numpy as jnp
from jax import lax
from jax.experimental import pallas as pl
from jax.experimental.pallas import tpu as pltpu
from jax.experimental.pallas import tpu_sc as plsc

N_LNC, N_MI, N_M = 10000, 5000, 20000
N_TOT = N_LNC + N_MI + N_M
OFF = {"lnc": 0, "mi": N_LNC, "m": N_LNC + N_MI}
FEAT = 256
OUT = 128
E = 50000
NB = 4
NR = 6

NCORES = 2
NS = 16
W = 128
NPCOL = FEAT // W
CHUNK = 128
EPT = 3200
E_PAD = NS * EPT
NCHUNK = EPT // CHUNK
NRACC = 10240
RPT = NRACC // NS
ZROWS = RPT // 16
NSEG = 4
SEGMENTS = (
    dict(rels=(1, 5), srct=("mi", "m"), dst_lo=0, n=N_LNC),
    dict(rels=(0, 3), srct=("lnc", "m"), dst_lo=0, n=N_MI),
    dict(rels=(2, 4), srct=("mi", "lnc"), dst_lo=0, n=10000),
    dict(rels=(2, 4), srct=("mi", "lnc"), dst_lo=10000, n=10000),
)
R_BLK = 1000
SEG_STARTS = (10, 15, 25)


def _seg_body(with_deg, *refs):
    if with_deg:
        (table, ones_hbm, srcidx, dstidx, zeros_hbm,
         s_out, deg_out, acc, srci_all, dsti_all, rows0, rows1, zbuf,
         gsem0, gsem1, zsem) = refs
    else:
        (table, srcidx, dstidx, zeros_hbm,
         s_out, acc, srci_all, dsti_all, rows0, rows1, zbuf, gsem0, gsem1,
         zsem) = refs
    c = lax.axis_index("c")
    s = lax.axis_index("s")
    rbase = s * RPT

    pltpu.sync_copy(zeros_hbm, zbuf)

    def zero_acc():
        for k in range(16):
            pltpu.async_copy(zbuf, acc.at[pl.ds(rbase + k * ZROWS, ZROWS)],
                             zsem)

    def zero_wait():
        for k in range(16):
            pltpu.make_async_copy(
                zbuf, acc.at[pl.ds(rbase + k * ZROWS, ZROWS)], zsem).wait()

    def run_pass(src_slab, dst_slab, out_at):
        zero_acc()
        pltpu.sync_copy(src_slab, srci_all)
        pltpu.sync_copy(dst_slab, dsti_all)

        def issue(j, rows, sem):
            pltpu.async_copy(table.at[srci_all.at[j]], rows, sem)

        def wait_scat(j, rows, sem):
            pltpu.make_async_copy(table.at[srci_all.at[j]], rows, sem).wait()
            pltpu.sync_copy(rows, acc.at[dsti_all.at[j]], add=True)

        issue(0, rows0, gsem0)
        issue(1, rows1, gsem1)
        zero_wait()
        plsc.subcore_barrier()
        assert NCHUNK % 2 == 1 and NCHUNK >= 3

        def pbody(t, carry):
            j0 = 2 * t
            wait_scat(j0, rows0, gsem0)
            issue(j0 + 2, rows0, gsem0)
            wait_scat(j0 + 1, rows1, gsem1)
            issue(j0 + 3, rows1, gsem1)
            return carry

        lax.fori_loop(0, (NCHUNK - 3) // 2, pbody, 0)
        wait_scat(NCHUNK - 3, rows0, gsem0)
        issue(NCHUNK - 1, rows0, gsem0)
        wait_scat(NCHUNK - 2, rows1, gsem1)
        wait_scat(NCHUNK - 1, rows0, gsem0)
        plsc.subcore_barrier()
        pltpu.sync_copy(acc.at[pl.ds(rbase, RPT)], out_at)

    def run_deg_pass(dst_slab, out_at):
        zero_acc()
        pltpu.sync_copy(dst_slab, dsti_all)
        pltpu.sync_copy(ones_hbm, rows0)
        zero_wait()
        plsc.subcore_barrier()

        def dbody(j, carry):
            pltpu.async_copy(rows0, acc.at[dsti_all.at[j]], gsem0, add=True)
            return carry

        lax.fori_loop(0, NCHUNK, dbody, 0)

        def dwait(j, carry):
            pltpu.make_async_copy(rows0, acc.at[dsti_all.at[j]],
                                  gsem0).wait()
            return carry

        lax.fori_loop(0, NCHUNK, dwait, 0)
        plsc.subcore_barrier()
        pltpu.sync_copy(acc.at[pl.ds(rbase, RPT)], out_at)

    for g in range(NSEG):
        for p in range(NPCOL):
            run_pass(
                srcidx.at[c, g, p, s],
                dstidx.at[c, g, s],
                s_out.at[c, g, p, pl.ds(rbase, RPT)],
            )
        if with_deg:
            run_deg_pass(
                dstidx.at[c, g, s],
                deg_out.at[c, g, pl.ds(rbase, RPT)],
            )


@functools.cache
def _make_seg_kernel(with_deg):
    out_type = [jax.ShapeDtypeStruct((NCORES, NSEG, NPCOL, NRACC, W),
                                     jnp.float32)]
    if with_deg:
        out_type.append(
            jax.ShapeDtypeStruct((NCORES, NSEG, NRACC, W), jnp.float32))
    scratch = (
        pltpu.VMEM_SHARED((NRACC, W), jnp.float32),
        pltpu.VMEM((NCHUNK, CHUNK), jnp.int32),
        pltpu.VMEM((NCHUNK, CHUNK), jnp.int32),
        pltpu.VMEM((CHUNK, W), jnp.float32),
        pltpu.VMEM((CHUNK, W), jnp.float32),
        pltpu.VMEM((ZROWS, W), jnp.float32),
        pltpu.SemaphoreType.DMA,
        pltpu.SemaphoreType.DMA,
        pltpu.SemaphoreType.DMA,
    )
    mesh = plsc.VectorSubcoreMesh(core_axis_name="c", subcore_axis_name="s")
    body = functools.partial(_seg_body, with_deg)
    return pl.kernel(body, out_type=tuple(out_type), mesh=mesh,
                     scratch_types=scratch,
                     name="segsum_deg" if with_deg else "segsum")


def _combine_w(coeff1, basis1, coeff2, basis2):
    def body(c1_ref, b1_ref, c2_ref, b2_ref, w1_ref, w2_ref):
        for c_ref, b_ref, w_ref in ((c1_ref, b1_ref, w1_ref),
                                    (c2_ref, b2_ref, w2_ref)):
            for r in range(NR):
                acc = c_ref[r, 0] * b_ref[0]
                for b in range(1, NB):
                    acc = acc + c_ref[r, b] * b_ref[b]
                w_ref[r] = acc

    out = jax.ShapeDtypeStruct((NR, FEAT, FEAT), jnp.float32)
    return pl.pallas_call(
        body,
        in_specs=[pl.BlockSpec(memory_space=pltpu.SMEM),
                  pl.BlockSpec((NB, FEAT, FEAT), lambda: (0, 0, 0)),
                  pl.BlockSpec(memory_space=pltpu.SMEM),
                  pl.BlockSpec((NB, FEAT, FEAT), lambda: (0, 0, 0))],
        out_specs=[pl.BlockSpec((NR, FEAT, FEAT), lambda: (0, 0, 0))] * 2,
        out_shape=[out, out],
    )(coeff1, basis1, coeff2, basis2)


def _seg_of(i):
    s = jnp.int32(0)
    for b in SEG_STARTS:
        s = s + (i >= b).astype(jnp.int32)
    return s


def _blk_of(i):
    off = jnp.int32(0)
    starts = (0,) + SEG_STARTS
    for k in range(1, NSEG):
        off = off + (i >= starts[k]).astype(jnp.int32) * (
            starts[k] - starts[k - 1])
    return i - off


def _tc_layer(h_all, s_all, deg_all, wp_all, loop_w, bias, dec):
    def bdot(x, y):
        return jnp.dot(x.astype(jnp.bfloat16), y.astype(jnp.bfloat16),
                       preferred_element_type=jnp.float32)

    def body(h_ref, s_ref, deg_ref, wp_ref, loop_ref, bias_ref, *rest):
        o_ref = rest[-1]
        acc = bdot(h_ref[...], loop_ref[...])
        for a in range(2):
            d = deg_ref[a, 0][:, 0:1]
            inv = 1.0 / jnp.maximum(d, 1.0)
            sa = jnp.concatenate([s_ref[a, 0, p] for p in range(NPCOL)],
                                 axis=1)
            acc = acc + bdot(sa * inv, wp_ref[0, a])
        x = jnp.maximum(acc + bias_ref[...], 0.0)
        if dec is None:
            o_ref[...] = x
        else:
            dw1_ref, db1_ref, dw2_ref, db2_ref = rest[:-1]
            y = jnp.maximum(bdot(x, dw1_ref[...]) + db1_ref[...], 0.0)
            o_ref[...] = bdot(y, dw2_ref[...]) + db2_ref[...]

    in_specs = [
        pl.BlockSpec((R_BLK, FEAT), lambda i: (i, 0)),
        pl.BlockSpec((NCORES, 1, NPCOL, R_BLK, W),
                     lambda i: (0, _seg_of(i), 0, _blk_of(i), 0)),
        pl.BlockSpec((NCORES, 1, R_BLK, 8),
                     lambda i: (0, _seg_of(i), _blk_of(i), 0)),
        pl.BlockSpec((1, NCORES, FEAT, FEAT),
                     lambda i: (_seg_of(i), 0, 0, 0)),
        pl.BlockSpec((FEAT, FEAT), lambda i: (0, 0)),
        pl.BlockSpec((1, FEAT), lambda i: (0, 0)),
    ]
    args = [h_all, s_all, deg_all, wp_all, loop_w, bias]
    out_w = FEAT
    if dec is not None:
        dw1, db1, dw2, db2 = dec
        in_specs += [
            pl.BlockSpec((FEAT, FEAT), lambda i: (0, 0)),
            pl.BlockSpec((1, FEAT), lambda i: (0, 0)),
            pl.BlockSpec((FEAT, OUT), lambda i: (0, 0)),
            pl.BlockSpec((1, OUT), lambda i: (0, 0)),
        ]
        args += [dw1, db1, dw2, db2]
        out_w = OUT

    return pl.pallas_call(
        body,
        grid=(N_TOT // R_BLK,),
        in_specs=in_specs,
        out_specs=pl.BlockSpec((R_BLK, out_w), lambda i: (i, 0)),
        out_shape=jax.ShapeDtypeStruct((N_TOT, out_w), jnp.float32),
    )(*args)


def kernel(h_lnc, h_mi, h_m, src0, dst0, src1, dst1, src2, dst2, src3, dst3,
           src4, dst4, src5, dst5, basis1, coeff1, loop1, bias1, basis2,
           coeff2, loop2, bias2, dec_W1, dec_b1, dec_W2, dec_b2):
    srcs = [src0, src1, src2, src3, src4, src5]
    dsts = [dst0, dst1, dst2, dst3, dst4, dst5]

    W1s, W2s = _combine_w(coeff1, basis1, coeff2, basis2)

    epad = E_PAD - E
    eidx = jnp.arange(E_PAD, dtype=jnp.int32)
    spread = eidx % 2048
    trash = (NRACC - 16) + (eidx & 15)
    src_segs, dst_segs = [], []
    for seg in SEGMENTS:
        src_pair, dst_pair = [], []
        for r, st in zip(seg["rels"], seg["srct"]):
            sp = jnp.concatenate(
                [srcs[r] + OFF[st], jnp.zeros((epad,), jnp.int32)])
            dp = jnp.concatenate(
                [dsts[r], jnp.full((epad,), jnp.int32(1 << 28))])
            lo = seg["dst_lo"]
            ok = (dp >= lo) & (dp < lo + seg["n"])
            dst_pair.append(jnp.where(ok, dp - lo, trash))
            src_pair.append(jnp.where(ok, sp, spread) * NPCOL)
        src_segs.append(jnp.stack(src_pair))
        dst_segs.append(jnp.stack(dst_pair))
    src_base = jnp.stack(src_segs, axis=1)
    srcidx = (jnp.stack([src_base + p for p in range(NPCOL)], axis=2)
              .reshape(NCORES, NSEG, NPCOL, NS, NCHUNK, CHUNK))
    dstidx = jnp.stack(dst_segs, axis=1).reshape(
        NCORES, NSEG, NS, NCHUNK, CHUNK)

    ones_tab = jnp.ones((CHUNK, W), jnp.float32)
    zeros_w = jnp.zeros((ZROWS, W), jnp.float32)

    wp_all = {}
    for li, Ws in ((0, W1s), (1, W2s)):
        wp_all[li] = jnp.stack(
            [jnp.stack([Ws[seg["rels"][0]], Ws[seg["rels"][1]]])
             for seg in SEGMENTS])

    h_all = jnp.concatenate([h_lnc, h_mi, h_m], axis=0)
    deg_all = None
    for layer in (0, 1):
        table = h_all.reshape(N_TOT * NPCOL, W)
        if layer == 0:
            s_all, deg_fat = _make_seg_kernel(True)(
                table, ones_tab, srcidx, dstidx, zeros_w)
            deg_all = deg_fat[:, :, :, :8]
        else:
            (s_all,) = _make_seg_kernel(False)(table, srcidx, dstidx, zeros_w)
        loop_w = loop1 if layer == 0 else loop2
        bias = (bias1 if layer == 0 else bias2).reshape(1, FEAT)
        dec = (None if layer == 0 else
               (dec_W1, dec_b1.reshape(1, FEAT), dec_W2,
                dec_b2.reshape(1, OUT)))
        h_all = _tc_layer(h_all, s_all, deg_all, wp_all[layer], loop_w,
                          bias, dec)
    return h_all
